# Initial kernel scaffold; baseline (speedup 1.0000x reference)
import jax
import jax.numpy as jnp
from jax.experimental import pallas as pl


def kernel(hidden_states, sim_matrix, gates, w1, w2):
    raise NotImplementedError("write your pallas kernel here")


# speedup vs baseline: 1.5663x; 1.5663x over previous
import functools
import math

import jax
import jax.numpy as jnp
from jax.experimental import pallas as pl
from jax.experimental.pallas import tpu as pltpu


_INV_SQRT2 = 1.0 / math.sqrt(2.0)


def _router_body(x_ref, sim_ref, gates_ref, w_ref):
    x = x_ref[...]
    s = sim_ref[...]
    xnorm = jnp.sqrt(jnp.sum(x * x, axis=1, keepdims=True))
    xn = x / jnp.maximum(xnorm, 1e-12)
    snorm = jnp.sqrt(jnp.sum(s * s, axis=0, keepdims=True))
    sn = s / jnp.maximum(snorm, 1e-12)
    logits = jnp.dot(xn, sn)
    thr = jax.nn.sigmoid(gates_ref[...])
    a = jnp.maximum(logits - thr, 0.0)
    active = a > 0.0
    amax = jnp.max(a, axis=1, keepdims=True)
    ex = jnp.where(active, jnp.exp(a - amax), 0.0)
    tot = jnp.sum(ex, axis=1, keepdims=True)
    w_ref[...] = ex / jnp.where(tot > 0.0, tot, 1.0)


def _ffn_body(x_ref, wgt_ref, w1_ref, w2_ref, out_ref):
    e = pl.program_id(1)
    i = pl.program_id(2)

    @pl.when((e == 0) & (i == 0))
    def _():
        out_ref[...] = jnp.zeros_like(out_ref)

    xb = x_ref[...]
    w1b = w1_ref[0]
    h = jax.lax.dot_general(xb, w1b, (((1,), (1,)), ((), ())))
    h = 0.5 * h * (1.0 + jax.lax.erf(h * _INV_SQRT2))
    wgt = wgt_ref[...]
    lane = jax.lax.broadcasted_iota(jnp.int32, wgt.shape, 1)
    wcol = jnp.sum(jnp.where(lane == e, wgt, 0.0), axis=1, keepdims=True)
    hw = h * wcol
    w2b = w2_ref[0]
    out_ref[...] += jax.lax.dot_general(hw, w2b, (((1,), (1,)), ((), ())))


def kernel(hidden_states, sim_matrix, gates, w1, w2):
    B, T, C = hidden_states.shape
    E, I, _ = w1.shape
    x = hidden_states.reshape(T, C)

    T_BLK = 256
    I_BLK = 512
    n_tb = T // T_BLK
    n_i = I // I_BLK

    wgt = pl.pallas_call(
        _router_body,
        grid=(n_tb,),
        in_specs=[
            pl.BlockSpec((T_BLK, C), lambda tb: (tb, 0)),
            pl.BlockSpec((C, E), lambda tb: (0, 0)),
            pl.BlockSpec((1, E), lambda tb: (0, 0)),
        ],
        out_specs=pl.BlockSpec((T_BLK, E), lambda tb: (tb, 0)),
        out_shape=jax.ShapeDtypeStruct((T, E), jnp.float32),
    )(x, sim_matrix, gates.reshape(1, E))

    out = pl.pallas_call(
        _ffn_body,
        grid=(n_tb, E, n_i),
        in_specs=[
            pl.BlockSpec((T_BLK, C), lambda tb, e, i: (tb, 0)),
            pl.BlockSpec((T_BLK, E), lambda tb, e, i: (tb, 0)),
            pl.BlockSpec((1, I_BLK, C), lambda tb, e, i: (e, i, 0)),
            pl.BlockSpec((1, C, I_BLK), lambda tb, e, i: (e, 0, i)),
        ],
        out_specs=pl.BlockSpec((T_BLK, C), lambda tb, e, i: (tb, 0)),
        out_shape=jax.ShapeDtypeStruct((T, C), jnp.float32),
        compiler_params=pltpu.CompilerParams(
            dimension_semantics=("parallel", "arbitrary", "arbitrary"),
        ),
    )(x, wgt, w1, w2)

    return out.reshape(B, T, C)


# speedup vs baseline: 1.9013x; 1.2139x over previous
import functools
import math

import jax
import jax.numpy as jnp
from jax.experimental import pallas as pl
from jax.experimental.pallas import tpu as pltpu


_INV_SQRT2 = 1.0 / math.sqrt(2.0)


def _router_body(x_ref, sim_ref, gates_ref, w_ref):
    x = x_ref[...]
    s = sim_ref[...]
    xnorm = jnp.sqrt(jnp.sum(x * x, axis=1, keepdims=True))
    xn = x / jnp.maximum(xnorm, 1e-12)
    snorm = jnp.sqrt(jnp.sum(s * s, axis=0, keepdims=True))
    sn = s / jnp.maximum(snorm, 1e-12)
    logits = jnp.dot(xn, sn)
    thr = jax.nn.sigmoid(gates_ref[...])
    a = jnp.maximum(logits - thr, 0.0)
    active = a > 0.0
    amax = jnp.max(a, axis=1, keepdims=True)
    ex = jnp.where(active, jnp.exp(a - amax), 0.0)
    tot = jnp.sum(ex, axis=1, keepdims=True)
    w_ref[...] = ex / jnp.where(tot > 0.0, tot, 1.0)


def _ffn_body(counts_ref, idx_ref, x_ref, wgt_ref, w1_ref, w2_ref, out_ref,
              xg_ref, acc_ref, *, T_BLK, n_i):
    e = pl.program_id(0)
    i = pl.program_id(1)
    tb = pl.program_id(2)
    T = x_ref.shape[0]

    @pl.when((e == 0) & (i == 0) & (tb == 0))
    def _():
        out_ref[...] = jnp.zeros_like(out_ref)

    cnt = counts_ref[e]
    base = tb * T_BLK

    @pl.when(base < cnt)
    def _():
        @pl.when(i == 0)
        def _():
            def gather_row(r, carry):
                t = idx_ref[e * T + base + r]
                xg_ref[pl.ds(base + r, 1), :] = x_ref[pl.ds(t, 1), :]
                return carry
            jax.lax.fori_loop(0, T_BLK, gather_row, 0, unroll=8)

        xb = xg_ref[pl.ds(base, T_BLK), :]
        w1b = w1_ref[0]
        h = jax.lax.dot_general(xb, w1b, (((1,), (1,)), ((), ())))
        h = 0.5 * h * (1.0 + jax.lax.erf(h * _INV_SQRT2))
        hw = h * wgt_ref[0, pl.ds(base, T_BLK), :]
        w2b = w2_ref[0]
        contrib = jax.lax.dot_general(hw, w2b, (((1,), (1,)), ((), ())))

        @pl.when(i == 0)
        def _():
            acc_ref[pl.ds(base, T_BLK), :] = contrib

        @pl.when(i > 0)
        def _():
            acc_ref[pl.ds(base, T_BLK), :] += contrib

        @pl.when(i == n_i - 1)
        def _():
            def scatter_row(r, carry):
                t = idx_ref[e * T + base + r]
                out_ref[pl.ds(t, 1), :] += acc_ref[pl.ds(base + r, 1), :]
                return carry
            jax.lax.fori_loop(0, T_BLK, scatter_row, 0, unroll=8)


def kernel(hidden_states, sim_matrix, gates, w1, w2):
    B, T, C = hidden_states.shape
    E, I, _ = w1.shape
    x = hidden_states.reshape(T, C)

    T_BLK = 256
    I_BLK = 512
    n_tb = T // T_BLK
    n_i = I // I_BLK

    wgt = pl.pallas_call(
        _router_body,
        grid=(n_tb,),
        in_specs=[
            pl.BlockSpec((T_BLK, C), lambda tb: (tb, 0)),
            pl.BlockSpec((C, E), lambda tb: (0, 0)),
            pl.BlockSpec((1, E), lambda tb: (0, 0)),
        ],
        out_specs=pl.BlockSpec((T_BLK, E), lambda tb: (tb, 0)),
        out_shape=jax.ShapeDtypeStruct((T, E), jnp.float32),
    )(x, sim_matrix, gates.reshape(1, E))

    active = wgt > 0.0
    counts = jnp.sum(active, axis=0).astype(jnp.int32)
    idx = jnp.argsort(~active, axis=0, stable=True).T.astype(jnp.int32)

    wgt_c = jnp.take_along_axis(wgt.T, idx, axis=1)

    out = pl.pallas_call(
        functools.partial(_ffn_body, T_BLK=T_BLK, n_i=n_i),
        grid_spec=pltpu.PrefetchScalarGridSpec(
            num_scalar_prefetch=2,
            grid=(E, n_i, n_tb),
            in_specs=[
                pl.BlockSpec((T, C), lambda e, i, tb, c_r, x_r: (0, 0)),
                pl.BlockSpec((1, T, 1), lambda e, i, tb, c_r, x_r: (e, 0, 0)),
                pl.BlockSpec((1, I_BLK, C), lambda e, i, tb, c_r, x_r: (e, i, 0)),
                pl.BlockSpec((1, C, I_BLK), lambda e, i, tb, c_r, x_r: (e, 0, i)),
            ],
            out_specs=pl.BlockSpec((T, C), lambda e, i, tb, c_r, x_r: (0, 0)),
            scratch_shapes=[
                pltpu.VMEM((T, C), jnp.float32),
                pltpu.VMEM((T, C), jnp.float32),
            ],
        ),
        out_shape=jax.ShapeDtypeStruct((T, C), jnp.float32),
        compiler_params=pltpu.CompilerParams(
            dimension_semantics=("arbitrary", "arbitrary", "arbitrary"),
        ),
    )(counts, idx.reshape(E * T), x, wgt_c.reshape(E, T, 1), w1, w2)

    return out.reshape(B, T, C)
